# baseline (speedup 1.0000x reference)
import functools

import jax
import jax.numpy as jnp
from jax import lax
from jax.experimental import pallas as pl
from jax.experimental.pallas import tpu as pltpu
from jax.experimental.pallas import tpu_sc as plsc

_B = 4096
_T = 32
_D = 64
_P = 7
_BT = _B * _T
_NSETS = 6 * _P
_N_ROWS = _NSETS * _BT
_NW = 32
_SPW = _B
_K = 128
_NCH = _SPW // _K
_N_ENTITY = 100000

_BB = 128
_NB = _B // _BB
_BBT = _BB * _T


@functools.cache
def _make_sc_gather():
    @functools.partial(
        pl.kernel,
        mesh=plsc.VectorSubcoreMesh(core_axis_name="c", subcore_axis_name="s"),
        out_type=[
            jax.ShapeDtypeStruct((_N_ROWS, _D), jnp.float32),
            jax.ShapeDtypeStruct((_B, _D), jnp.float32),
        ],
        scratch_types=[
            pltpu.VMEM((_SPW,), jnp.int32),
            pltpu.VMEM((_K, _D), jnp.float32),
            pltpu.VMEM((_K, _D), jnp.float32),
            pltpu.SemaphoreType.DMA,
            pltpu.SemaphoreType.DMA,
        ],
        compiler_params=pltpu.CompilerParams(use_tc_tiling_on_sc=False),
    )
    def _sc_gather(idx_all, tab, out_main, out_items,
                   idx_v, rows0, rows1, sem0, sem1):
        wid = lax.axis_index("s") * 2 + lax.axis_index("c")
        rows = (rows0, rows1)
        sems = (sem0, sem1)

        def per_set(s, carry):
            base = s * _BT + wid * _SPW
            pltpu.sync_copy(idx_all.at[pl.ds(base, _SPW)], idx_v)
            cps = [None, None]
            cps[0] = pltpu.async_copy(
                tab.at[idx_v.at[pl.ds(0, _K)]], rows[0], sems[0])
            for c in range(_NCH):
                b = c % 2
                if c + 1 < _NCH:
                    b2 = (c + 1) % 2
                    cps[b2] = pltpu.async_copy(
                        tab.at[idx_v.at[pl.ds((c + 1) * _K, _K)]],
                        rows[b2], sems[b2])
                cps[b].wait()
                pltpu.sync_copy(rows[b], out_main.at[pl.ds(base + c * _K, _K)])
            return carry

        lax.fori_loop(0, _NSETS, per_set, 0)
        ipw = _B // _NW
        ib = wid * ipw
        pltpu.sync_copy(idx_all.at[pl.ds(_N_ROWS + ib, ipw)],
                        idx_v.at[pl.ds(0, ipw)])
        cp = pltpu.async_copy(
            tab.at[idx_v.at[pl.ds(0, ipw)]], rows0, sem0)
        cp.wait()
        pltpu.sync_copy(rows0, out_items.at[pl.ds(ib, ipw)])

    return _sc_gather


def _tc_body(g0, g1, g2, g3, g4, g5, items_ref, w1_ref, w2_ref,
             out_ref, acc_u, acc_v):
    p = pl.program_id(1)
    w1h = w1_ref[:_D, :]
    w1p = w1_ref[_D:, :]
    w2 = w2_ref[0, :]

    e00 = g0[...]
    e01 = g1[...]
    r10 = g2[...]
    r11 = g3[...]
    t0 = g4[...]
    t1 = g5[...]

    def att_out(h, pp, t):
        s1 = jax.nn.sigmoid(
            jnp.dot(h, w1h, preferred_element_type=jnp.float32)
            + jnp.dot(pp, w1p, preferred_element_type=jnp.float32))
        a = jax.nn.sigmoid(
            jnp.sum(s1.reshape(_BB, _T, _D) * w2[None, None, :], axis=-1))
        ea = jnp.exp(a)
        att = ea / jnp.sum(ea, axis=-1, keepdims=True)
        return jnp.sum(t.reshape(_BB, _T, _D) * att[:, :, None], axis=1)

    out0 = att_out(e00, r10, t0)
    out1 = att_out(e00 + e01, r10 * r11, t1)
    mean0 = jnp.sum(e00.reshape(_BB, _T, _D), axis=1) * (1.0 / _T)
    base = mean0 + out0 + out1

    @pl.when(p == 0)
    def _():
        acc_u[...] = jnp.zeros_like(acc_u)
        acc_v[...] = jnp.zeros_like(acc_v)

    u_w = jnp.where(p < 4, 1.0, 0.0)
    v_w = jnp.where(p < 4, 0.0, jnp.where(p == 4, 2.0, 1.0))
    item_w = jnp.where(p == 4, 2.0, 0.0)
    acc_u[...] += u_w * base
    acc_v[...] += v_w * base + item_w * items_ref[...]

    @pl.when(p == _P - 1)
    def _():
        out_ref[...] = jax.nn.sigmoid(
            jnp.sum(acc_u[...] * acc_v[...], axis=-1))


def _tc_compute(g_main, g_items, W1, W2row):
    set_specs = [
        pl.BlockSpec((_BBT, _D), lambda i, p, j=j: ((p * 6 + j) * _NB + i, 0))
        for j in range(6)
    ]
    return pl.pallas_call(
        _tc_body,
        grid=(_NB, _P),
        in_specs=set_specs + [
            pl.BlockSpec((_BB, _D), lambda i, p: (i, 0)),
            pl.BlockSpec((2 * _D, _D), lambda i, p: (0, 0)),
            pl.BlockSpec((1, _D), lambda i, p: (0, 0)),
        ],
        out_specs=pl.BlockSpec((_BB,), lambda i, p: (i,)),
        out_shape=jax.ShapeDtypeStruct((_B,), jnp.float32),
        scratch_shapes=[
            pltpu.VMEM((_BB, _D), jnp.float32),
            pltpu.VMEM((_BB, _D), jnp.float32),
        ],
    )(g_main, g_main, g_main, g_main, g_main, g_main,
      g_items, W1, W2row)


def kernel(items, kg_init_triple_set, ddi_potential_triple_set,
           kg_potential_triple_set, ddi_origin_triple_set,
           kg_init_triple_set1, ddi_potential_triple_set1,
           kg_potential_triple_set1, ddi_origin_triple_set1,
           embeddings_0, embeddings_1, entity_emb, relation_emb,
           W_att1, W_att2):
    procs = [kg_init_triple_set, kg_potential_triple_set,
             kg_init_triple_set1, kg_potential_triple_set1,
             ddi_potential_triple_set, ddi_potential_triple_set1,
             ddi_origin_triple_set1]
    off = jnp.array([0, _N_ENTITY, 0], jnp.int32).reshape(3, 1, 1, 1)
    idx_all = jnp.concatenate(
        [(ts + off).reshape(-1) for ts in procs] + [items])
    tab = jnp.concatenate([entity_emb, relation_emb])
    g_main, g_items = _make_sc_gather()(idx_all, tab)
    return _tc_compute(g_main, g_items, W_att1, W_att2.reshape(1, _D))


# speedup vs baseline: 1.0803x; 1.0803x over previous
import functools

import jax
import jax.numpy as jnp
from jax import lax
from jax.experimental import pallas as pl
from jax.experimental.pallas import tpu as pltpu
from jax.experimental.pallas import tpu_sc as plsc

_B = 4096
_T = 32
_D = 64
_P = 7
_NSETS = 6 * _P
_NW = 32
_K = 128
_N_ENTITY = 100000

_H = 4
_BH = _B // _H
_BTH = _BH * _T
_N_ROWS_H = _NSETS * _BTH
_SPW = _BTH // _NW
_NCH = _SPW // _K

_BB = 128
_NB_H = _BH // _BB
_BBT = _BB * _T


@functools.cache
def _make_sc_gather():
    @functools.partial(
        pl.kernel,
        mesh=plsc.VectorSubcoreMesh(core_axis_name="c", subcore_axis_name="s"),
        out_type=[
            jax.ShapeDtypeStruct((_N_ROWS_H, _D), jnp.float32),
            jax.ShapeDtypeStruct((_BH, _D), jnp.float32),
        ],
        scratch_types=[
            pltpu.VMEM((_SPW,), jnp.int32),
            pltpu.VMEM((_K, _D), jnp.float32),
            pltpu.VMEM((_K, _D), jnp.float32),
            pltpu.SemaphoreType.DMA,
            pltpu.SemaphoreType.DMA,
        ],
        compiler_params=pltpu.CompilerParams(use_tc_tiling_on_sc=False),
    )
    def _sc_gather(idx_all, tab, out_main, out_items,
                   idx_v, rows0, rows1, sem0, sem1):
        wid = lax.axis_index("s") * 2 + lax.axis_index("c")
        rows = (rows0, rows1)
        sems = (sem0, sem1)

        def per_set(s, carry):
            base = s * _BTH + wid * _SPW
            pltpu.sync_copy(idx_all.at[pl.ds(base, _SPW)], idx_v)
            cps = [None, None]
            cps[0] = pltpu.async_copy(
                tab.at[idx_v.at[pl.ds(0, _K)]], rows[0], sems[0])
            for c in range(_NCH):
                b = c % 2
                if c + 1 < _NCH:
                    b2 = (c + 1) % 2
                    cps[b2] = pltpu.async_copy(
                        tab.at[idx_v.at[pl.ds((c + 1) * _K, _K)]],
                        rows[b2], sems[b2])
                cps[b].wait()
                pltpu.sync_copy(rows[b], out_main.at[pl.ds(base + c * _K, _K)])
            return carry

        lax.fori_loop(0, _NSETS, per_set, 0)
        ipw = _BH // _NW
        ib = wid * ipw
        pltpu.sync_copy(idx_all.at[pl.ds(_N_ROWS_H + ib, ipw)],
                        idx_v.at[pl.ds(0, ipw)])
        cp = pltpu.async_copy(
            tab.at[idx_v.at[pl.ds(0, ipw)]], rows0.at[pl.ds(0, ipw)], sem0)
        cp.wait()
        pltpu.sync_copy(rows0.at[pl.ds(0, ipw)], out_items.at[pl.ds(ib, ipw)])

    return _sc_gather


def _tc_body(g_hbm, items_ref, w1_ref, w2_ref, out_ref, buf, sems, acc_u, acc_v):
    n = pl.program_id(0)
    p = lax.rem(n, _P)
    w1h = w1_ref[:_D, :]
    w1p = w1_ref[_D:, :]
    w2 = w2_ref[0, :]

    def issue(m, slot):
        i = lax.div(m, _P)
        pp = lax.rem(m, _P)
        for j in range(6):
            off = (pp * 6 + j) * _BTH + i * _BBT
            pltpu.make_async_copy(
                g_hbm.at[pl.ds(off, _BBT), :], buf.at[slot, j],
                sems.at[slot, j]).start()

    @pl.when(n == 0)
    def _():
        issue(0, 0)

    slot = lax.rem(n, 2)

    @pl.when(n + 1 < _NB_H * _P)
    def _():
        issue(n + 1, 1 - slot)

    for j in range(6):
        pltpu.make_async_copy(
            g_hbm.at[pl.ds(0, _BBT), :], buf.at[slot, j],
            sems.at[slot, j]).wait()

    e00 = buf[slot, 0]
    e01 = buf[slot, 1]
    r10 = buf[slot, 2]
    r11 = buf[slot, 3]
    t0 = buf[slot, 4]
    t1 = buf[slot, 5]

    def att_out(h, pp, t):
        s1 = jax.nn.sigmoid(
            jnp.dot(h, w1h, preferred_element_type=jnp.float32)
            + jnp.dot(pp, w1p, preferred_element_type=jnp.float32))
        a = jax.nn.sigmoid(
            jnp.sum(s1.reshape(_BB, _T, _D) * w2[None, None, :], axis=-1))
        ea = jnp.exp(a)
        att = ea / jnp.sum(ea, axis=-1, keepdims=True)
        return jnp.sum(t.reshape(_BB, _T, _D) * att[:, :, None], axis=1)

    out0 = att_out(e00, r10, t0)
    out1 = att_out(e00 + e01, r10 * r11, t1)
    mean0 = jnp.sum(e00.reshape(_BB, _T, _D), axis=1) * (1.0 / _T)
    base = mean0 + out0 + out1

    @pl.when(p == 0)
    def _():
        acc_u[...] = jnp.zeros_like(acc_u)
        acc_v[...] = jnp.zeros_like(acc_v)

    u_w = jnp.where(p < 4, 1.0, 0.0)
    v_w = jnp.where(p < 4, 0.0, jnp.where(p == 4, 2.0, 1.0))
    item_w = jnp.where(p == 4, 2.0, 0.0)
    acc_u[...] += u_w * base
    acc_v[...] += v_w * base + item_w * items_ref[...]

    @pl.when(p == _P - 1)
    def _():
        out_ref[...] = jax.nn.sigmoid(
            jnp.sum(acc_u[...] * acc_v[...], axis=-1))


def _tc_compute(g_main, g_items, W1, W2row):
    return pl.pallas_call(
        _tc_body,
        grid=(_NB_H * _P,),
        in_specs=[
            pl.BlockSpec(memory_space=pl.ANY),
            pl.BlockSpec((_BB, _D), lambda n: (n // _P, 0)),
            pl.BlockSpec((2 * _D, _D), lambda n: (0, 0)),
            pl.BlockSpec((1, _D), lambda n: (0, 0)),
        ],
        out_specs=pl.BlockSpec((_BB,), lambda n: (n // _P,)),
        out_shape=jax.ShapeDtypeStruct((_BH,), jnp.float32),
        scratch_shapes=[
            pltpu.VMEM((2, 6, _BBT, _D), jnp.float32),
            pltpu.SemaphoreType.DMA((2, 6)),
            pltpu.VMEM((_BB, _D), jnp.float32),
            pltpu.VMEM((_BB, _D), jnp.float32),
        ],
    )(g_main, g_items, W1, W2row)


def kernel(items, kg_init_triple_set, ddi_potential_triple_set,
           kg_potential_triple_set, ddi_origin_triple_set,
           kg_init_triple_set1, ddi_potential_triple_set1,
           kg_potential_triple_set1, ddi_origin_triple_set1,
           embeddings_0, embeddings_1, entity_emb, relation_emb,
           W_att1, W_att2):
    procs = [kg_init_triple_set, kg_potential_triple_set,
             kg_init_triple_set1, kg_potential_triple_set1,
             ddi_potential_triple_set, ddi_potential_triple_set1,
             ddi_origin_triple_set1]
    off = jnp.array([0, _N_ENTITY, 0], jnp.int32).reshape(3, 1, 1, 1)
    tab = jnp.concatenate([entity_emb, relation_emb])
    W2row = W_att2.reshape(1, _D)
    sc = _make_sc_gather()
    outs = []
    for h in range(_H):
        sl = slice(h * _BH, (h + 1) * _BH)
        idx_h = jnp.concatenate(
            [(ts[:, :, sl, :] + off).reshape(-1) for ts in procs]
            + [items[sl]])
        g_main, g_items = sc(idx_h, tab)
        outs.append(_tc_compute(g_main, g_items, W_att1, W2row))
    return jnp.concatenate(outs)
